# Initial kernel scaffold; baseline (speedup 1.0000x reference)
import jax
import jax.numpy as jnp
from jax.experimental import pallas as pl


def kernel(net_input, user_emb, rest_emb, W1, b1, W2, b2, W3, b3):
    raise NotImplementedError("write your pallas kernel here")


# speedup vs baseline: 1.5393x; 1.5393x over previous
import jax
import jax.numpy as jnp
from jax import lax
from jax.experimental import pallas as pl
from jax.experimental.pallas import tpu as pltpu
from jax.experimental.pallas import tpu_sc as plsc

_B = 16384
_E = 64
_HIST = 50
_NC, _NS = 2, 16
_NW = _NC * _NS
_BPW = _B // _NW
_SPC = 2
_ROWS = _SPC * (_HIST + 1)
_CH = _BPW // _SPC
_NBUF = 2

_LANES = 4


def _accum_chunk(rows_ref, racc_ref, c):
    scale = jnp.float32(1.0 / _HIST)
    for s in range(_SPC):
        off = 1 + s * (_HIST + 1)
        for v in range(_LANES):
            sl = pl.ds(v * 16, 16)
            acc = rows_ref[off, sl]
            for j in range(1, _HIST):
                acc = acc + rows_ref[off + j, sl]
            racc_ref[_SPC * c + s, sl] = acc * scale


def _gather_pool_kernel(pair_hbm, uidx_hbm, uemb_hbm, remb_hbm,
                        u_out, r_out,
                        nidx_v, uidx_v, urows_v, racc_v,
                        rows0, rows1, usem, sem0, sem1):
    wid = lax.axis_index("s") * _NC + lax.axis_index("c")
    base = wid * _BPW

    pltpu.sync_copy(pair_hbm.at[pl.ds(wid * _CH, _CH)], nidx_v)
    pltpu.sync_copy(uidx_hbm.at[pl.ds(wid * 4, 4)], uidx_v)

    for j in range(4):
        pltpu.async_copy(uemb_hbm.at[uidx_v.at[j]],
                         urows_v.at[pl.ds(j * 128, 128)], usem)

    rows = (rows0, rows1)
    sems = (sem0, sem1)

    for b in range(_NBUF):
        pltpu.async_copy(remb_hbm.at[nidx_v.at[b]], rows[b], sems[b])

    @pl.loop(0, _CH, step=_NBUF)
    def _(t):
        for b in range(_NBUF):
            c = t + b
            pltpu.make_async_copy(remb_hbm.at[nidx_v.at[b]],
                                  rows[b], sems[b]).wait()
            _accum_chunk(rows[b], racc_v, c)

            @pl.when(c + _NBUF < _CH)
            def _():
                pltpu.async_copy(remb_hbm.at[nidx_v.at[c + _NBUF]],
                                 rows[b], sems[b])

    for j in range(4):
        pltpu.make_async_copy(uemb_hbm.at[uidx_v.at[j]],
                              urows_v.at[pl.ds(j * 128, 128)], usem).wait()
    pltpu.sync_copy(urows_v, u_out.at[pl.ds(base, _BPW)])
    pltpu.sync_copy(racc_v, r_out.at[pl.ds(base, _BPW)])


_gather_pool = pl.kernel(
    _gather_pool_kernel,
    out_type=(jax.ShapeDtypeStruct((_B, _E), jnp.float32),
              jax.ShapeDtypeStruct((_B, _E), jnp.float32)),
    mesh=plsc.VectorSubcoreMesh(core_axis_name="c", subcore_axis_name="s",
                                num_cores=_NC, num_subcores=_NS),
    compiler_params=pltpu.CompilerParams(use_tc_tiling_on_sc=False),
    scratch_types=[
        pltpu.VMEM((_CH, _ROWS), jnp.int32),
        pltpu.VMEM((4, 128), jnp.int32),
        pltpu.VMEM((_BPW, _E), jnp.float32),
        pltpu.VMEM((_BPW, _E), jnp.float32),
        pltpu.VMEM((_ROWS, _E), jnp.float32),
        pltpu.VMEM((_ROWS, _E), jnp.float32),
        pltpu.SemaphoreType.DMA,
        pltpu.SemaphoreType.DMA,
        pltpu.SemaphoreType.DMA,
    ],
)


_BT = 2048


def _mlp_body(u_ref, r_ref, w1_ref, b1_ref, w2_ref, b2_ref, w3_ref, b3_ref,
              o_ref):
    x = jnp.concatenate([u_ref[...], r_ref[...]], axis=1)
    h1 = jnp.maximum(
        jnp.dot(x, w1_ref[...], preferred_element_type=jnp.float32)
        + b1_ref[...], 0.0)
    h2 = jnp.maximum(
        jnp.dot(h1, w2_ref[...], preferred_element_type=jnp.float32)
        + b2_ref[...], 0.0)
    y = jnp.sum(h2 * w3_ref[...], axis=1, keepdims=True) + b3_ref[...]
    o_ref[...] = jax.nn.sigmoid(y)


def _mlp(u, r, w1t, b1r, w2t, b2r, w3r, b3r):
    return pl.pallas_call(
        _mlp_body,
        grid=(_B // _BT,),
        in_specs=[
            pl.BlockSpec((_BT, _E), lambda i: (i, 0)),
            pl.BlockSpec((_BT, _E), lambda i: (i, 0)),
            pl.BlockSpec((2 * _E, 256), lambda i: (0, 0)),
            pl.BlockSpec((1, 256), lambda i: (0, 0)),
            pl.BlockSpec((256, 128), lambda i: (0, 0)),
            pl.BlockSpec((1, 128), lambda i: (0, 0)),
            pl.BlockSpec((1, 128), lambda i: (0, 0)),
            pl.BlockSpec((1, 1), lambda i: (0, 0)),
        ],
        out_specs=pl.BlockSpec((_BT, 1), lambda i: (i, 0)),
        out_shape=jax.ShapeDtypeStruct((_B, 1), jnp.float32),
    )(u, r, w1t, b1r, w2t, b2r, w3r, b3r)


def kernel(net_input, user_emb, rest_emb, W1, b1, W2, b2, W3, b3):
    pair_idx = net_input.reshape(_B // _SPC, _ROWS)
    user_idx = net_input[:, 0].reshape(_NW * 4, 128)
    u, r = _gather_pool(pair_idx, user_idx, user_emb, rest_emb)
    return _mlp(u, r,
                W1.T, b1.reshape(1, -1),
                W2.T, b2.reshape(1, -1),
                W3.reshape(1, -1), b3.reshape(1, 1))
